# baseline (speedup 1.0000x reference)
import functools

import jax
import jax.numpy as jnp
from jax import lax
from jax.experimental import pallas as pl
from jax.experimental.pallas import tpu as pltpu
from jax.experimental.pallas import tpu_sc as plsc

D_M = 1024
N_POS = 4096
N_BATCH = 4
_INFO = plsc.get_sparse_core_info()
_NC, _NS, _L = _INFO.num_cores, _INFO.num_subcores, _INFO.num_lanes
_NW = _NC * _NS
_P_PER_W = N_POS // _NW
_P_CHUNK = 8
_N_CHUNKS = _P_PER_W // _P_CHUNK
_R_CHUNK = _P_CHUNK * N_BATCH
_NVEC = D_M // _L


def _sc_body(x_hbm, emb_hbm, src_hbm, out_hbm, idx_v, xbuf, encbuf, sem):
    wid = lax.axis_index("s") * _NC + lax.axis_index("c")
    base_p = wid * _P_PER_W
    pltpu.sync_copy(src_hbm.at[pl.ds(base_p, _P_PER_W)], idx_v)

    def chunk(c, carry):
        row0 = (base_p + c * _P_CHUNK) * N_BATCH
        pltpu.sync_copy(x_hbm.at[pl.ds(row0, _R_CHUNK)], xbuf)
        pltpu.async_copy(
            emb_hbm.at[idx_v.at[pl.ds(c * _P_CHUNK, _P_CHUNK)]], encbuf, sem
        ).wait()

        def col(j, carry2):
            off = j * _L
            for p in range(_P_CHUNK):
                ev = encbuf[p, pl.ds(off, _L)]
                for b in range(N_BATCH):
                    r = p * N_BATCH + b
                    xbuf[r, pl.ds(off, _L)] = xbuf[r, pl.ds(off, _L)] + ev
            return carry2

        lax.fori_loop(0, _NVEC, col, 0)
        pltpu.sync_copy(xbuf, out_hbm.at[pl.ds(row0, _R_CHUNK)])
        return carry

    lax.fori_loop(0, _N_CHUNKS, chunk, 0)


@functools.partial(jax.jit, static_argnums=())
def _sc_call(xf, emb, src_ids):
    mesh = plsc.VectorSubcoreMesh(core_axis_name="c", subcore_axis_name="s")
    f = pl.kernel(
        _sc_body,
        mesh=mesh,
        out_type=jax.ShapeDtypeStruct((N_POS * N_BATCH, D_M), jnp.float32),
        scratch_types=[
            pltpu.VMEM((_P_PER_W,), jnp.int32),
            pltpu.VMEM((_R_CHUNK, D_M), jnp.float32),
            pltpu.VMEM((_P_CHUNK, D_M), jnp.float32),
            pltpu.SemaphoreType.DMA,
        ],
    )
    return f(xf, emb, src_ids)


def kernel(x, emb, src_ids):
    total, batch, d = x.shape
    xf = x.reshape(total * batch, d)
    out = _sc_call(xf, emb, src_ids)
    return out.reshape(total, batch, d)


# speedup vs baseline: 1.0151x; 1.0151x over previous
import functools

import jax
import jax.numpy as jnp
from jax import lax
from jax.experimental import pallas as pl
from jax.experimental.pallas import tpu as pltpu
from jax.experimental.pallas import tpu_sc as plsc

D_M = 1024
N_POS = 4096
N_BATCH = 4
_INFO = plsc.get_sparse_core_info()
_NC, _NS, _L = _INFO.num_cores, _INFO.num_subcores, _INFO.num_lanes
_NW = _NC * _NS
_P_PER_W = N_POS // _NW
_P_CHUNK = 8
_N_CHUNKS = _P_PER_W // _P_CHUNK
_R_CHUNK = _P_CHUNK * N_BATCH
_NVEC = D_M // _L


def _sc_body(x_hbm, emb_hbm, src_hbm, out_hbm, idx_v,
             xbuf0, xbuf1, encbuf0, encbuf1,
             isem0, isem1, osem0, osem1, gsem0, gsem1):
    xbufs = (xbuf0, xbuf1)
    encbufs = (encbuf0, encbuf1)
    isems = (isem0, isem1)
    osems = (osem0, osem1)
    gsems = (gsem0, gsem1)

    wid = lax.axis_index("s") * _NC + lax.axis_index("c")
    base_p = wid * _P_PER_W
    pltpu.sync_copy(src_hbm.at[pl.ds(base_p, _P_PER_W)], idx_v)

    def in_copy(c):
        b = c % 2
        row0 = (base_p + c * _P_CHUNK) * N_BATCH
        return pltpu.make_async_copy(
            x_hbm.at[pl.ds(row0, _R_CHUNK)], xbufs[b], isems[b])

    def gather(c):
        b = c % 2
        return pltpu.make_async_copy(
            emb_hbm.at[idx_v.at[pl.ds(c * _P_CHUNK, _P_CHUNK)]],
            encbufs[b], gsems[b])

    def out_copy(c):
        b = c % 2
        row0 = (base_p + c * _P_CHUNK) * N_BATCH
        return pltpu.make_async_copy(
            xbufs[b], out_hbm.at[pl.ds(row0, _R_CHUNK)], osems[b])

    in_copy(0).start()
    gather(0).start()
    for c in range(_N_CHUNKS):
        b = c % 2
        in_copy(c).wait()
        gather(c).wait()
        if c + 1 < _N_CHUNKS:
            if c >= 1:
                out_copy(c - 1).wait()
            in_copy(c + 1).start()
            gather(c + 1).start()

        xbuf, encbuf = xbufs[b], encbufs[b]

        def col(j, carry2):
            off = j * _L
            for p in range(_P_CHUNK):
                ev = encbuf[p, pl.ds(off, _L)]
                for bb in range(N_BATCH):
                    r = p * N_BATCH + bb
                    xbuf[r, pl.ds(off, _L)] = xbuf[r, pl.ds(off, _L)] + ev
            return carry2

        lax.fori_loop(0, _NVEC, col, 0)
        out_copy(c).start()
    out_copy(_N_CHUNKS - 2).wait()
    out_copy(_N_CHUNKS - 1).wait()


@functools.partial(jax.jit, static_argnums=())
def _sc_call(xf, emb, src_ids):
    mesh = plsc.VectorSubcoreMesh(core_axis_name="c", subcore_axis_name="s")
    f = pl.kernel(
        _sc_body,
        mesh=mesh,
        out_type=jax.ShapeDtypeStruct((N_POS * N_BATCH, D_M), jnp.float32),
        scratch_types=[
            pltpu.VMEM((_P_PER_W,), jnp.int32),
            pltpu.VMEM((_R_CHUNK, D_M), jnp.float32),
            pltpu.VMEM((_R_CHUNK, D_M), jnp.float32),
            pltpu.VMEM((_P_CHUNK, D_M), jnp.float32),
            pltpu.VMEM((_P_CHUNK, D_M), jnp.float32),
            pltpu.SemaphoreType.DMA,
            pltpu.SemaphoreType.DMA,
            pltpu.SemaphoreType.DMA,
            pltpu.SemaphoreType.DMA,
            pltpu.SemaphoreType.DMA,
            pltpu.SemaphoreType.DMA,
        ],
    )
    return f(xf, emb, src_ids)


def kernel(x, emb, src_ids):
    total, batch, d = x.shape
    xf = x.reshape(total * batch, d)
    out = _sc_call(xf, emb, src_ids)
    return out.reshape(total, batch, d)


# speedup vs baseline: 4.0679x; 4.0074x over previous
import functools

import jax
import jax.numpy as jnp
from jax import lax
from jax.experimental import pallas as pl
from jax.experimental.pallas import tpu as pltpu
from jax.experimental.pallas import tpu_sc as plsc

D_M = 1024
N_POS = 4096
N_BATCH = 4
N_SRC = 4
_INFO = plsc.get_sparse_core_info()
_NC, _NS, _L = _INFO.num_cores, _INFO.num_subcores, _INFO.num_lanes
_NW = _NC * _NS
_P_PER_W = N_POS // _NW
_P_CHUNK = 4
_N_CHUNKS = _P_PER_W // _P_CHUNK
_NVEC = D_M // _L
_NBUF = 5
_ID = 3
_OD = 2


def _sc_body(x_hbm, emb_hbm, ids_rep_hbm, out_hbm, *refs):
    xbufs = refs[:_NBUF]
    idx_v = refs[_NBUF]
    emb_v = refs[_NBUF + 1]
    isems = refs[_NBUF + 2:2 * _NBUF + 2]
    osems = refs[2 * _NBUF + 2:3 * _NBUF + 2]

    wid = lax.axis_index("s") * _NC + lax.axis_index("c")
    base_p = wid * _P_PER_W
    pltpu.sync_copy(ids_rep_hbm.at[pl.ds(base_p, _P_PER_W)], idx_v)
    pltpu.sync_copy(emb_hbm, emb_v)

    def in_copy(c):
        b = c % _NBUF
        return pltpu.make_async_copy(
            x_hbm.at[pl.ds(base_p + c * _P_CHUNK, _P_CHUNK)], xbufs[b], isems[b])

    def out_copy(c):
        b = c % _NBUF
        return pltpu.make_async_copy(
            xbufs[b], out_hbm.at[pl.ds(base_p + c * _P_CHUNK, _P_CHUNK)],
            osems[b])

    for c in range(_ID):
        in_copy(c).start()
    for c in range(_N_CHUNKS):
        b = c % _NBUF
        in_copy(c).wait()
        xbuf = xbufs[b]

        idvecs = [idx_v[c * _P_CHUNK + p, :] for p in range(_P_CHUNK)]

        def col(j, carry, xbuf=xbuf, idvecs=idvecs):
            off = pl.ds(j * _L, _L)
            evs = [emb_v[s, off] for s in range(N_SRC)]
            for p in range(_P_CHUNK):
                iv = idvecs[p]
                ev = evs[N_SRC - 1]
                for s in range(N_SRC - 2, -1, -1):
                    ev = jnp.where(iv == s, evs[s], ev)
                for bb in range(N_BATCH):
                    xbuf[p, bb, off] = xbuf[p, bb, off] + ev
            return carry

        lax.fori_loop(0, _NVEC, col, 0)
        out_copy(c).start()
        if c >= _OD:
            out_copy(c - _OD).wait()
        if c + _ID < _N_CHUNKS:
            in_copy(c + _ID).start()
    for c in range(_N_CHUNKS - _OD, _N_CHUNKS):
        out_copy(c).wait()


@functools.partial(jax.jit, static_argnums=())
def _sc_call(x, emb, src_ids):
    mesh = plsc.VectorSubcoreMesh(core_axis_name="c", subcore_axis_name="s")
    scratch = [pltpu.VMEM((_P_CHUNK, N_BATCH, D_M), jnp.float32)
               for _ in range(_NBUF)]
    scratch += [
        pltpu.VMEM((_P_PER_W, _L), jnp.int32),
        pltpu.VMEM((N_SRC, D_M), jnp.float32),
    ]
    scratch += [pltpu.SemaphoreType.DMA for _ in range(2 * _NBUF)]
    f = pl.kernel(
        _sc_body,
        mesh=mesh,
        out_type=jax.ShapeDtypeStruct((N_POS, N_BATCH, D_M), jnp.float32),
        scratch_types=scratch,
    )
    ids_rep = jnp.broadcast_to(src_ids[:, None], (N_POS, _L))
    return f(x, emb, ids_rep)


def kernel(x, emb, src_ids):
    return _sc_call(x, emb, src_ids)


# speedup vs baseline: 4.9833x; 1.2250x over previous
import functools

import jax
import jax.numpy as jnp
from jax import lax
from jax.experimental import pallas as pl
from jax.experimental.pallas import tpu as pltpu
from jax.experimental.pallas import tpu_sc as plsc

D_M = 1024
N_POS = 4096
N_BATCH = 4
N_SRC = 4
_INFO = plsc.get_sparse_core_info()
_NC, _NS, _L = _INFO.num_cores, _INFO.num_subcores, _INFO.num_lanes
_NW = _NC * _NS
_P_PER_W = N_POS // _NW
_P_CHUNK = 4
_N_CHUNKS = _P_PER_W // _P_CHUNK
_NVEC = D_M // _L
_NBUF = 5
_ID = 3
_OD = 2


def _sc_body(x_hbm, emb_hbm, ids_rep_hbm, out_hbm, *refs):
    xbufs = refs[:_NBUF]
    idx_v = refs[_NBUF]
    emb_v = refs[_NBUF + 1]
    isems = refs[_NBUF + 2:2 * _NBUF + 2]
    osems = refs[2 * _NBUF + 2:3 * _NBUF + 2]

    wid = lax.axis_index("s") * _NC + lax.axis_index("c")
    base_p = wid * _P_PER_W
    pltpu.sync_copy(ids_rep_hbm.at[pl.ds(base_p, _P_PER_W)], idx_v)
    pltpu.sync_copy(emb_hbm, emb_v)

    def in_copy(c):
        b = c % _NBUF
        return pltpu.make_async_copy(
            x_hbm.at[pl.ds(base_p + c * _P_CHUNK, _P_CHUNK)], xbufs[b], isems[b])

    def out_copy(c):
        b = c % _NBUF
        return pltpu.make_async_copy(
            xbufs[b], out_hbm.at[pl.ds(base_p + c * _P_CHUNK, _P_CHUNK)],
            osems[b])

    for c in range(_ID):
        in_copy(c).start()
    for c in range(_N_CHUNKS):
        b = c % _NBUF
        in_copy(c).wait()
        xbuf = xbufs[b]

        idvecs = [idx_v[c * _P_CHUNK + p, :] for p in range(_P_CHUNK)]

        def col(j, carry, xbuf=xbuf, idvecs=idvecs):
            off = pl.ds(j * _L, _L)
            evs = [emb_v[s, off] for s in range(N_SRC)]
            for p in range(_P_CHUNK):
                iv = idvecs[p]
                ev = evs[N_SRC - 1]
                for s in range(N_SRC - 2, -1, -1):
                    ev = jnp.where(iv == s, evs[s], ev)
                for bb in range(N_BATCH):
                    xbuf[p, bb, off] = xbuf[p, bb, off] + ev
            return carry

        lax.fori_loop(0, _NVEC, col, 0)
        out_copy(c).start()
        if c >= _OD:
            out_copy(c - _OD).wait()
        if c + _ID < _N_CHUNKS:
            in_copy(c + _ID).start()
    for c in range(_N_CHUNKS - _OD, _N_CHUNKS):
        out_copy(c).wait()


@functools.partial(jax.jit, static_argnums=())
def _sc_call(x, emb, src_ids):
    mesh = plsc.VectorSubcoreMesh(core_axis_name="c", subcore_axis_name="s")
    scratch = [pltpu.VMEM((_P_CHUNK, N_BATCH, D_M), jnp.float32)
               for _ in range(_NBUF)]
    scratch += [
        pltpu.VMEM((_P_PER_W, _L), jnp.int32),
        pltpu.VMEM((N_SRC, D_M), jnp.float32),
    ]
    scratch += [pltpu.SemaphoreType.DMA for _ in range(2 * _NBUF)]
    f = pl.kernel(
        _sc_body,
        mesh=mesh,
        out_type=jax.ShapeDtypeStruct((N_POS, N_BATCH, D_M), jnp.float32),
        scratch_types=scratch,
    )
    ids_rep = jnp.broadcast_to(src_ids[:, None], (N_POS, _L))
    return f(x, emb, ids_rep)


def kernel(x, emb, src_ids):
    return _sc_call(x, emb, src_ids)


_BP = 128


def _tc_body(ids_ref, emb_ref, x_ref, o_ref):
    ids = ids_ref[...]
    emb = emb_ref[...]
    n_sources, d = emb.shape
    enc = jnp.zeros((ids.shape[0], d), jnp.float32)
    for s in range(n_sources):
        enc = jnp.where(ids == s, emb[s].reshape(1, d), enc)
    o_ref[...] = x_ref[...] + enc[:, None, :]


def _tc_call(x, emb, src_ids, bp):
    total, batch, d = x.shape
    grid = total // bp
    ids2 = src_ids.reshape(total, 1)
    return pl.pallas_call(
        _tc_body,
        grid=(grid,),
        in_specs=[
            pl.BlockSpec((bp, 1), lambda i: (i, 0)),
            pl.BlockSpec(emb.shape, lambda i: (0, 0)),
            pl.BlockSpec((bp, batch, d), lambda i: (i, 0, 0)),
        ],
        out_specs=pl.BlockSpec((bp, batch, d), lambda i: (i, 0, 0)),
        out_shape=jax.ShapeDtypeStruct(x.shape, x.dtype),
    )(ids2, emb, x)


def kernel(x, emb, src_ids):
    return _tc_call(x, emb, src_ids, _BP)


# speedup vs baseline: 5.7840x; 1.1607x over previous
import functools

import jax
import jax.numpy as jnp
from jax import lax
from jax.experimental import pallas as pl
from jax.experimental.pallas import tpu as pltpu
from jax.experimental.pallas import tpu_sc as plsc

D_M = 1024
N_POS = 4096
N_BATCH = 4
N_SRC = 4
_INFO = plsc.get_sparse_core_info()
_NC, _NS, _L = _INFO.num_cores, _INFO.num_subcores, _INFO.num_lanes
_NW = _NC * _NS
_P_PER_W = N_POS // _NW
_P_CHUNK = 4
_N_CHUNKS = _P_PER_W // _P_CHUNK
_NVEC = D_M // _L
_NBUF = 5
_ID = 3
_OD = 2


def _sc_body(x_hbm, emb_hbm, ids_rep_hbm, out_hbm, *refs):
    xbufs = refs[:_NBUF]
    idx_v = refs[_NBUF]
    emb_v = refs[_NBUF + 1]
    isems = refs[_NBUF + 2:2 * _NBUF + 2]
    osems = refs[2 * _NBUF + 2:3 * _NBUF + 2]

    wid = lax.axis_index("s") * _NC + lax.axis_index("c")
    base_p = wid * _P_PER_W
    pltpu.sync_copy(ids_rep_hbm.at[pl.ds(base_p, _P_PER_W)], idx_v)
    pltpu.sync_copy(emb_hbm, emb_v)

    def in_copy(c):
        b = c % _NBUF
        return pltpu.make_async_copy(
            x_hbm.at[pl.ds(base_p + c * _P_CHUNK, _P_CHUNK)], xbufs[b], isems[b])

    def out_copy(c):
        b = c % _NBUF
        return pltpu.make_async_copy(
            xbufs[b], out_hbm.at[pl.ds(base_p + c * _P_CHUNK, _P_CHUNK)],
            osems[b])

    for c in range(_ID):
        in_copy(c).start()
    for c in range(_N_CHUNKS):
        b = c % _NBUF
        in_copy(c).wait()
        xbuf = xbufs[b]

        idvecs = [idx_v[c * _P_CHUNK + p, :] for p in range(_P_CHUNK)]

        def col(j, carry, xbuf=xbuf, idvecs=idvecs):
            off = pl.ds(j * _L, _L)
            evs = [emb_v[s, off] for s in range(N_SRC)]
            for p in range(_P_CHUNK):
                iv = idvecs[p]
                ev = evs[N_SRC - 1]
                for s in range(N_SRC - 2, -1, -1):
                    ev = jnp.where(iv == s, evs[s], ev)
                for bb in range(N_BATCH):
                    xbuf[p, bb, off] = xbuf[p, bb, off] + ev
            return carry

        lax.fori_loop(0, _NVEC, col, 0)
        out_copy(c).start()
        if c >= _OD:
            out_copy(c - _OD).wait()
        if c + _ID < _N_CHUNKS:
            in_copy(c + _ID).start()
    for c in range(_N_CHUNKS - _OD, _N_CHUNKS):
        out_copy(c).wait()


@functools.partial(jax.jit, static_argnums=())
def _sc_call(x, emb, src_ids):
    mesh = plsc.VectorSubcoreMesh(core_axis_name="c", subcore_axis_name="s")
    scratch = [pltpu.VMEM((_P_CHUNK, N_BATCH, D_M), jnp.float32)
               for _ in range(_NBUF)]
    scratch += [
        pltpu.VMEM((_P_PER_W, _L), jnp.int32),
        pltpu.VMEM((N_SRC, D_M), jnp.float32),
    ]
    scratch += [pltpu.SemaphoreType.DMA for _ in range(2 * _NBUF)]
    f = pl.kernel(
        _sc_body,
        mesh=mesh,
        out_type=jax.ShapeDtypeStruct((N_POS, N_BATCH, D_M), jnp.float32),
        scratch_types=scratch,
    )
    ids_rep = jnp.broadcast_to(src_ids[:, None], (N_POS, _L))
    return f(x, emb, ids_rep)


def kernel(x, emb, src_ids):
    return _sc_call(x, emb, src_ids)


_BP = 512


def _tc_body(ids_ref, emb_ref, x_ref, o_ref):
    ids = ids_ref[...]
    emb = emb_ref[...]
    n_sources, d = emb.shape
    enc = jnp.zeros((ids.shape[0], d), jnp.float32)
    for s in range(n_sources):
        enc = jnp.where(ids == s, emb[s].reshape(1, d), enc)
    o_ref[...] = x_ref[...] + enc[:, None, :]


def _tc_call(x, emb, src_ids, bp):
    total, batch, d = x.shape
    grid = total // bp
    ids2 = src_ids.reshape(total, 1)
    return pl.pallas_call(
        _tc_body,
        grid=(grid,),
        in_specs=[
            pl.BlockSpec((bp, 1), lambda i: (i, 0)),
            pl.BlockSpec(emb.shape, lambda i: (0, 0)),
            pl.BlockSpec((bp, batch, d), lambda i: (i, 0, 0)),
        ],
        out_specs=pl.BlockSpec((bp, batch, d), lambda i: (i, 0, 0)),
        out_shape=jax.ShapeDtypeStruct(x.shape, x.dtype),
    )(ids2, emb, x)


def kernel(x, emb, src_ids):
    return _tc_call(x, emb, src_ids, _BP)
